# baseline (speedup 1.0000x reference)
import functools

import jax
import jax.numpy as jnp
from jax import lax
from jax.experimental import pallas as pl
from jax.experimental.pallas import tpu as pltpu
from jax.experimental.pallas import tpu_sc as plsc

_NSIGMA = 3.0
_PATCH = 8
_CH = 2000

_NW = 32
_CD = 25
_NCHUNK = 100000 // _NW // _CD


def _prep_body(c_ref, s_ref, ch_ref, h_ref, par_ref, off_ref):
    c = c_ref[0]
    s = s_ref[0]
    inv_sqrt2 = 0.7071067811865476
    ir3 = lax.broadcasted_iota(jnp.int32, (3, 1), 0)
    h = jnp.where(ir3 == 0, h_ref[0], jnp.where(ir3 == 1, h_ref[1], h_ref[2]))

    offf = jnp.floor((c - _NSIGMA * s) / h)
    invs = inv_sqrt2 / s
    b0 = (offf * h - c) * invs
    step = h * invs

    ch = ch_ref[0]
    cdf_prev = 0.5 * (1.0 + lax.erf(b0))
    q0r, q1r, q2r = [], [], []
    for t in range(1, _PATCH + 1):
        cdf = 0.5 * (1.0 + lax.erf(b0 + float(t) * step))
        d = cdf - cdf_prev
        q0r.append(d[0:1] * ch)
        q1r.append(d[1:2])
        q2r.append(d[2:3])
        cdf_prev = cdf
    zeros8 = jnp.zeros((8, c.shape[1]), jnp.float32)
    p_t = jnp.concatenate(q0r + q1r + q2r + [zeros8], axis=0)
    eye = (lax.broadcasted_iota(jnp.int32, (32, 32), 0)
           == lax.broadcasted_iota(jnp.int32, (32, 32), 1)).astype(jnp.float32)
    par_ref[...] = lax.dot_general(p_t, eye, (((0,), (0,)), ((), ())),
                                   preferred_element_type=jnp.float32)
    off_ref[0] = offf.astype(jnp.int32)


def _tc_prep(c_t, s_t, ch_t, grid_spacing, n, grid):
    return pl.pallas_call(
        _prep_body,
        grid=(grid,),
        in_specs=[
            pl.BlockSpec((1, 3, _CH), lambda i: (i, 0, 0)),
            pl.BlockSpec((1, 3, _CH), lambda i: (i, 0, 0)),
            pl.BlockSpec((1, 1, _CH), lambda i: (i, 0, 0)),
            pl.BlockSpec(memory_space=pltpu.SMEM),
        ],
        out_specs=[
            pl.BlockSpec((_CH, 32), lambda i: (i, 0)),
            pl.BlockSpec((1, 3, _CH), lambda i: (i, 0, 0)),
        ],
        out_shape=[
            jax.ShapeDtypeStruct((n, 32), jnp.float32),
            jax.ShapeDtypeStruct((grid, 3, _CH), jnp.int32),
        ],
    )(c_t, s_t, ch_t, grid_spacing)


def _vgather(vec, idx):
    return lax.gather(
        vec, idx[:, None],
        lax.GatherDimensionNumbers(offset_dims=(),
                                   collapsed_slice_dims=(0,),
                                   start_index_map=(0,)),
        (1,), mode=lax.GatherScatterMode.PROMISE_IN_BOUNDS)


def _sc_raster_body(par_hbm, out_hbm, pbuf, obuf0, obuf1, sem0, sem1):
    nc = 2
    wid = lax.axis_index("s") * nc + lax.axis_index("c")
    base = wid * (_CD * _NCHUNK)

    iot = lax.iota(jnp.int32, 16)
    idx_q2rep = iot & 7
    idx_qp = [8 + 2 * u + (iot >> 3) for u in range(4)]
    idx_p = [iot * 0 + i for i in range(8)]

    def do_chunk(t, obuf, sem):
        s0 = base + t * _CD
        pltpu.sync_copy(par_hbm.at[pl.ds(s0 * 32, _CD * 32)], pbuf)

        def depo_body(d, carry):
            va = pbuf[pl.ds(d * 32, 16)]
            vb = pbuf[pl.ds(d * 32 + 16, 16)]
            q2rep = _vgather(vb, idx_q2rep)
            qp = [_vgather(va, idx_qp[u]) * q2rep for u in range(4)]
            for i in range(8):
                p = _vgather(va, idx_p[i])
                for u in range(4):
                    obuf[pl.ds(d * 512 + i * 64 + u * 16, 16)] = p * qp[u]
            return carry

        lax.fori_loop(0, _CD, depo_body, 0, unroll=1)
        pltpu.make_async_copy(
            obuf, out_hbm.at[pl.ds(s0 * 512, _CD * 512)], sem).start()

    def chunk_pair(p, carry):
        @pl.when(p >= 1)
        def _():
            pltpu.make_async_copy(
                obuf0, out_hbm.at[pl.ds(base * 512, _CD * 512)], sem0).wait()
            pltpu.make_async_copy(
                obuf1, out_hbm.at[pl.ds(base * 512, _CD * 512)], sem1).wait()
        do_chunk(p * 2, obuf0, sem0)
        do_chunk(p * 2 + 1, obuf1, sem1)
        return carry

    lax.fori_loop(0, _NCHUNK // 2, chunk_pair, 0, unroll=1)
    pltpu.make_async_copy(
        obuf0, out_hbm.at[pl.ds(base * 512, _CD * 512)], sem0).wait()
    do_chunk(_NCHUNK - 1, obuf0, sem0)
    pltpu.make_async_copy(
        obuf0, out_hbm.at[pl.ds(base * 512, _CD * 512)], sem0).wait()
    pltpu.make_async_copy(
        obuf1, out_hbm.at[pl.ds(base * 512, _CD * 512)], sem1).wait()


def _sc_raster(params_flat, n):
    mesh = plsc.VectorSubcoreMesh(core_axis_name="c", subcore_axis_name="s")
    kern = functools.partial(
        pl.kernel,
        out_type=jax.ShapeDtypeStruct((n * 512,), jnp.float32),
        mesh=mesh,
        scratch_types=[
            pltpu.VMEM((_CD * 32,), jnp.float32),
            pltpu.VMEM((_CD * 512,), jnp.float32),
            pltpu.VMEM((_CD * 512,), jnp.float32),
            pltpu.SemaphoreType.DMA,
            pltpu.SemaphoreType.DMA,
        ],
    )(_sc_raster_body)
    return kern(params_flat)


def kernel(sigma, time, charge, tail, grid_spacing, velocity):
    n = sigma.shape[0]
    grid = n // _CH
    c_t = jnp.stack([tail[:, 1], tail[:, 0], time]).reshape(3, grid, _CH)
    c_t = c_t.transpose(1, 0, 2)
    s_t = sigma.T.reshape(3, grid, _CH).transpose(1, 0, 2)
    ch_t = charge.reshape(grid, 1, _CH)
    params, offsets_t = _tc_prep(c_t, s_t, ch_t, grid_spacing, n, grid)
    rasters = _sc_raster(params.reshape(n * 32), n)
    offsets = offsets_t.transpose(1, 0, 2).reshape(3, n).T
    return rasters.reshape(n, _PATCH, _PATCH, _PATCH), offsets


# speedup vs baseline: 1.0017x; 1.0017x over previous
import functools

import jax
import jax.numpy as jnp
from jax import lax
from jax.experimental import pallas as pl
from jax.experimental.pallas import tpu as pltpu
from jax.experimental.pallas import tpu_sc as plsc

_NSIGMA = 3.0
_PATCH = 8
_CH = 2000

_NW = 32
_CD = 25
_NCHUNK = 100000 // _NW // _CD


def _prep_body(c_ref, s_ref, ch_ref, h_ref, par_ref, off_ref):
    c = c_ref[0]
    s = s_ref[0]
    inv_sqrt2 = 0.7071067811865476
    ir3 = lax.broadcasted_iota(jnp.int32, (3, 1), 0)
    h = jnp.where(ir3 == 0, h_ref[0], jnp.where(ir3 == 1, h_ref[1], h_ref[2]))

    offf = jnp.floor((c - _NSIGMA * s) / h)
    invs = inv_sqrt2 / s
    b0 = (offf * h - c) * invs
    step = h * invs

    ch = ch_ref[0]
    cdf_prev = 0.5 * (1.0 + lax.erf(b0))
    q0r, q1r, q2r = [], [], []
    for t in range(1, _PATCH + 1):
        cdf = 0.5 * (1.0 + lax.erf(b0 + float(t) * step))
        d = cdf - cdf_prev
        q0r.append(d[0:1] * ch)
        q1r.append(d[1:2])
        q2r.append(d[2:3])
        cdf_prev = cdf
    zeros8 = jnp.zeros((8, c.shape[1]), jnp.float32)
    p_t = jnp.concatenate(q0r + q1r + q2r + [zeros8], axis=0)
    eye = (lax.broadcasted_iota(jnp.int32, (32, 32), 0)
           == lax.broadcasted_iota(jnp.int32, (32, 32), 1)).astype(jnp.float32)
    par_ref[...] = lax.dot_general(p_t, eye, (((0,), (0,)), ((), ())),
                                   preferred_element_type=jnp.float32)
    off_ref[0] = offf.astype(jnp.int32)


def _tc_prep(c_t, s_t, ch_t, grid_spacing, n, grid):
    return pl.pallas_call(
        _prep_body,
        grid=(grid,),
        in_specs=[
            pl.BlockSpec((1, 3, _CH), lambda i: (i, 0, 0)),
            pl.BlockSpec((1, 3, _CH), lambda i: (i, 0, 0)),
            pl.BlockSpec((1, 1, _CH), lambda i: (i, 0, 0)),
            pl.BlockSpec(memory_space=pltpu.SMEM),
        ],
        out_specs=[
            pl.BlockSpec((_CH, 32), lambda i: (i, 0)),
            pl.BlockSpec((1, 3, _CH), lambda i: (i, 0, 0)),
        ],
        out_shape=[
            jax.ShapeDtypeStruct((n, 32), jnp.float32),
            jax.ShapeDtypeStruct((grid, 3, _CH), jnp.int32),
        ],
    )(c_t, s_t, ch_t, grid_spacing)


def _vgather(vec, idx):
    return lax.gather(
        vec, idx[:, None],
        lax.GatherDimensionNumbers(offset_dims=(),
                                   collapsed_slice_dims=(0,),
                                   start_index_map=(0,)),
        (1,), mode=lax.GatherScatterMode.PROMISE_IN_BOUNDS)


def _sc_raster_body(par_hbm, out_hbm, pbuf, obuf0, obuf1, sem0, sem1):
    nc = 2
    wid = lax.axis_index("s") * nc + lax.axis_index("c")
    base = wid * (_CD * _NCHUNK)

    iot = lax.iota(jnp.int32, 16)
    idx_q2rep = iot & 7
    idx_qp = [8 + 2 * u + (iot >> 3) for u in range(4)]
    idx_p = [iot * 0 + i for i in range(8)]

    def do_chunk(t, obuf, sem):
        s0 = base + t * _CD
        pltpu.sync_copy(par_hbm.at[pl.ds(s0 * 32, _CD * 32)], pbuf)

        @plsc.parallel_loop(0, _CD, 1, unroll=4)
        def _depo(d):
            va = pbuf[pl.ds(d * 32, 16)]
            vb = pbuf[pl.ds(d * 32 + 16, 16)]
            q2rep = _vgather(vb, idx_q2rep)
            qp = [_vgather(va, idx_qp[u]) * q2rep for u in range(4)]
            for i in range(8):
                p = _vgather(va, idx_p[i])
                for u in range(4):
                    obuf[pl.ds(d * 512 + i * 64 + u * 16, 16)] = p * qp[u]
        pltpu.make_async_copy(
            obuf, out_hbm.at[pl.ds(s0 * 512, _CD * 512)], sem).start()

    def chunk_pair(p, carry):
        @pl.when(p >= 1)
        def _():
            pltpu.make_async_copy(
                obuf0, out_hbm.at[pl.ds(base * 512, _CD * 512)], sem0).wait()
            pltpu.make_async_copy(
                obuf1, out_hbm.at[pl.ds(base * 512, _CD * 512)], sem1).wait()
        do_chunk(p * 2, obuf0, sem0)
        do_chunk(p * 2 + 1, obuf1, sem1)
        return carry

    lax.fori_loop(0, _NCHUNK // 2, chunk_pair, 0, unroll=1)
    pltpu.make_async_copy(
        obuf0, out_hbm.at[pl.ds(base * 512, _CD * 512)], sem0).wait()
    do_chunk(_NCHUNK - 1, obuf0, sem0)
    pltpu.make_async_copy(
        obuf0, out_hbm.at[pl.ds(base * 512, _CD * 512)], sem0).wait()
    pltpu.make_async_copy(
        obuf1, out_hbm.at[pl.ds(base * 512, _CD * 512)], sem1).wait()


def _sc_raster(params_flat, n):
    mesh = plsc.VectorSubcoreMesh(core_axis_name="c", subcore_axis_name="s")
    kern = functools.partial(
        pl.kernel,
        out_type=jax.ShapeDtypeStruct((n * 512,), jnp.float32),
        mesh=mesh,
        scratch_types=[
            pltpu.VMEM((_CD * 32,), jnp.float32),
            pltpu.VMEM((_CD * 512,), jnp.float32),
            pltpu.VMEM((_CD * 512,), jnp.float32),
            pltpu.SemaphoreType.DMA,
            pltpu.SemaphoreType.DMA,
        ],
    )(_sc_raster_body)
    return kern(params_flat)


def kernel(sigma, time, charge, tail, grid_spacing, velocity):
    n = sigma.shape[0]
    grid = n // _CH
    c_t = jnp.stack([tail[:, 1], tail[:, 0], time]).reshape(3, grid, _CH)
    c_t = c_t.transpose(1, 0, 2)
    s_t = sigma.T.reshape(3, grid, _CH).transpose(1, 0, 2)
    ch_t = charge.reshape(grid, 1, _CH)
    params, offsets_t = _tc_prep(c_t, s_t, ch_t, grid_spacing, n, grid)
    rasters = _sc_raster(params.reshape(n * 32), n)
    offsets = offsets_t.transpose(1, 0, 2).reshape(3, n).T
    return rasters.reshape(n, _PATCH, _PATCH, _PATCH), offsets


# speedup vs baseline: 7.7520x; 7.7390x over previous
import functools

import jax
import jax.numpy as jnp
from jax import lax
from jax.experimental import pallas as pl
from jax.experimental.pallas import tpu as pltpu
from jax.experimental.pallas import tpu_sc as plsc

_NSIGMA = 3.0
_PATCH = 8
_CH = 2000

_NW = 32
_CD = 40
_NCHUNK = 100000 // _CD
_JFULL = _NCHUNK // _NW
_EXTRA = _NCHUNK - _JFULL * _NW


def _prep_body(c_ref, s_ref, ch_ref, h_ref, par_ref, off_ref):
    c = c_ref[0]
    s = s_ref[0]
    inv_sqrt2 = 0.7071067811865476
    ir3 = lax.broadcasted_iota(jnp.int32, (3, 1), 0)
    h = jnp.where(ir3 == 0, h_ref[0], jnp.where(ir3 == 1, h_ref[1], h_ref[2]))

    offf = jnp.floor((c - _NSIGMA * s) / h)
    invs = inv_sqrt2 / s
    b0 = (offf * h - c) * invs
    step = h * invs

    ch = ch_ref[0]
    cdf_prev = 0.5 * (1.0 + lax.erf(b0))
    q0r, q1r, q2r = [], [], []
    for t in range(1, _PATCH + 1):
        cdf = 0.5 * (1.0 + lax.erf(b0 + float(t) * step))
        d = cdf - cdf_prev
        q0r.append(d[0:1] * ch)
        q1r.append(d[1:2])
        q2r.append(d[2:3])
        cdf_prev = cdf
    zeros8 = jnp.zeros((8, c.shape[1]), jnp.float32)
    p_t = jnp.concatenate(q0r + q1r + q2r + [zeros8], axis=0)
    eye = (lax.broadcasted_iota(jnp.int32, (32, 32), 0)
           == lax.broadcasted_iota(jnp.int32, (32, 32), 1)).astype(jnp.float32)
    par_ref[...] = lax.dot_general(p_t, eye, (((0,), (0,)), ((), ())),
                                   preferred_element_type=jnp.float32)
    off_ref[0] = offf.astype(jnp.int32)


def _tc_prep(c_t, s_t, ch_t, grid_spacing, n, grid):
    return pl.pallas_call(
        _prep_body,
        grid=(grid,),
        in_specs=[
            pl.BlockSpec((1, 3, _CH), lambda i: (i, 0, 0)),
            pl.BlockSpec((1, 3, _CH), lambda i: (i, 0, 0)),
            pl.BlockSpec((1, 1, _CH), lambda i: (i, 0, 0)),
            pl.BlockSpec(memory_space=pltpu.SMEM),
        ],
        out_specs=[
            pl.BlockSpec((_CH, 32), lambda i: (i, 0)),
            pl.BlockSpec((1, 3, _CH), lambda i: (i, 0, 0)),
        ],
        out_shape=[
            jax.ShapeDtypeStruct((n, 32), jnp.float32),
            jax.ShapeDtypeStruct((grid, 3, _CH), jnp.int32),
        ],
    )(c_t, s_t, ch_t, grid_spacing)


def _vgather(vec, idx):
    return lax.gather(
        vec, idx[:, None],
        lax.GatherDimensionNumbers(offset_dims=(),
                                   collapsed_slice_dims=(0,),
                                   start_index_map=(0,)),
        (1,), mode=lax.GatherScatterMode.PROMISE_IN_BOUNDS)


def _sc_raster_body(par_hbm, out_hbm, pbuf, obuf0, obuf1, sem0, sem1):
    nc = 2
    wid = lax.axis_index("s") * nc + lax.axis_index("c")

    iot = lax.iota(jnp.int32, 16)
    idx_q2rep = iot & 7
    idx_qp = [8 + 2 * u + (iot >> 3) for u in range(4)]
    idx_p = [iot * 0 + i for i in range(8)]

    def do_chunk(g, obuf, sem):
        s0 = g * _CD
        pltpu.sync_copy(par_hbm.at[pl.ds(s0, _CD)], pbuf)

        @plsc.parallel_loop(0, _CD, 1, unroll=4)
        def _depo(d):
            va = pbuf[d, pl.ds(0, 16)]
            vb = pbuf[d, pl.ds(16, 16)]
            q2rep = _vgather(vb, idx_q2rep)
            qp = [_vgather(va, idx_qp[u]) * q2rep for u in range(4)]
            for i in range(8):
                p = _vgather(va, idx_p[i])
                for u in range(4):
                    obuf[d, pl.ds(i * 64 + u * 16, 16)] = p * qp[u]
        pltpu.make_async_copy(
            obuf, out_hbm.at[pl.ds(s0, _CD)], sem).start()

    def chunk_pair(p, carry):
        @pl.when(p >= 1)
        def _():
            pltpu.make_async_copy(
                obuf0, out_hbm.at[pl.ds(0, _CD)], sem0).wait()
            pltpu.make_async_copy(
                obuf1, out_hbm.at[pl.ds(0, _CD)], sem1).wait()
        do_chunk((p * 2) * _NW + wid, obuf0, sem0)
        do_chunk((p * 2 + 1) * _NW + wid, obuf1, sem1)
        return carry

    lax.fori_loop(0, _JFULL // 2, chunk_pair, 0, unroll=1)

    @pl.when(wid < _EXTRA)
    def _():
        pltpu.make_async_copy(
            obuf0, out_hbm.at[pl.ds(0, _CD)], sem0).wait()
        do_chunk(_JFULL * _NW + wid, obuf0, sem0)

    pltpu.make_async_copy(
        obuf0, out_hbm.at[pl.ds(0, _CD)], sem0).wait()
    pltpu.make_async_copy(
        obuf1, out_hbm.at[pl.ds(0, _CD)], sem1).wait()


def _sc_raster(params_flat, n):
    mesh = plsc.VectorSubcoreMesh(core_axis_name="c", subcore_axis_name="s")
    kern = functools.partial(
        pl.kernel,
        out_type=jax.ShapeDtypeStruct((n, 512), jnp.float32),
        mesh=mesh,
        scratch_types=[
            pltpu.VMEM((_CD, 32), jnp.float32),
            pltpu.VMEM((_CD, 512), jnp.float32),
            pltpu.VMEM((_CD, 512), jnp.float32),
            pltpu.SemaphoreType.DMA,
            pltpu.SemaphoreType.DMA,
        ],
    )(_sc_raster_body)
    return kern(params_flat)


def kernel(sigma, time, charge, tail, grid_spacing, velocity):
    n = sigma.shape[0]
    grid = n // _CH
    c_t = jnp.stack([tail[:, 1], tail[:, 0], time]).reshape(3, grid, _CH)
    c_t = c_t.transpose(1, 0, 2)
    s_t = sigma.T.reshape(3, grid, _CH).transpose(1, 0, 2)
    ch_t = charge.reshape(grid, 1, _CH)
    params, offsets_t = _tc_prep(c_t, s_t, ch_t, grid_spacing, n, grid)
    rasters = _sc_raster(params, n)
    offsets = offsets_t.transpose(1, 0, 2).reshape(3, n).T
    return rasters.reshape(n, _PATCH, _PATCH, _PATCH), offsets


# speedup vs baseline: 9.7084x; 1.2524x over previous
import functools

import jax
import jax.numpy as jnp
from jax import lax
from jax.experimental import pallas as pl
from jax.experimental.pallas import tpu as pltpu
from jax.experimental.pallas import tpu_sc as plsc

_NSIGMA = 3.0
_PATCH = 8
_CH = 2000

_NW = 32
_CD = 40
_NCHUNK = 100000 // _CD
_JFULL = _NCHUNK // _NW
_EXTRA = _NCHUNK - _JFULL * _NW


def _prep_body(c_ref, s_ref, ch_ref, h_ref, par_ref, off_ref):
    c = c_ref[0]
    s = s_ref[0]
    inv_sqrt2 = 0.7071067811865476
    ir3 = lax.broadcasted_iota(jnp.int32, (3, 1), 0)
    h = jnp.where(ir3 == 0, h_ref[0], jnp.where(ir3 == 1, h_ref[1], h_ref[2]))

    offf = jnp.floor((c - _NSIGMA * s) / h)
    invs = inv_sqrt2 / s
    b0 = (offf * h - c) * invs
    step = h * invs

    ch = ch_ref[0]
    cdf_prev = 0.5 * (1.0 + lax.erf(b0))
    q0r, q1r, q2r = [], [], []
    for t in range(1, _PATCH + 1):
        cdf = 0.5 * (1.0 + lax.erf(b0 + float(t) * step))
        d = cdf - cdf_prev
        q0r.append(d[0:1] * ch)
        q1r.append(d[1:2])
        q2r.append(d[2:3])
        cdf_prev = cdf
    zeros8 = jnp.zeros((8, c.shape[1]), jnp.float32)
    p_t = jnp.concatenate(q0r + q1r + q2r + [zeros8], axis=0)
    eye = (lax.broadcasted_iota(jnp.int32, (32, 32), 0)
           == lax.broadcasted_iota(jnp.int32, (32, 32), 1)).astype(jnp.float32)
    par_ref[...] = lax.dot_general(p_t, eye, (((0,), (0,)), ((), ())),
                                   preferred_element_type=jnp.float32)
    off_ref[0] = offf.astype(jnp.int32)


def _tc_prep(c_t, s_t, ch_t, grid_spacing, n, grid):
    return pl.pallas_call(
        _prep_body,
        grid=(grid,),
        in_specs=[
            pl.BlockSpec((1, 3, _CH), lambda i: (i, 0, 0)),
            pl.BlockSpec((1, 3, _CH), lambda i: (i, 0, 0)),
            pl.BlockSpec((1, 1, _CH), lambda i: (i, 0, 0)),
            pl.BlockSpec(memory_space=pltpu.SMEM),
        ],
        out_specs=[
            pl.BlockSpec((_CH, 32), lambda i: (i, 0)),
            pl.BlockSpec((1, 3, _CH), lambda i: (i, 0, 0)),
        ],
        out_shape=[
            jax.ShapeDtypeStruct((n, 32), jnp.float32),
            jax.ShapeDtypeStruct((grid, 3, _CH), jnp.int32),
        ],
    )(c_t, s_t, ch_t, grid_spacing)


def _vgather(vec, idx):
    return lax.gather(
        vec, idx[:, None],
        lax.GatherDimensionNumbers(offset_dims=(),
                                   collapsed_slice_dims=(0,),
                                   start_index_map=(0,)),
        (1,), mode=lax.GatherScatterMode.PROMISE_IN_BOUNDS)


def _sc_raster_body(par_hbm, out_hbm, pbuf0, pbuf1, obuf0, obuf1,
                    sem0, sem1, psem0, psem1):
    nc = 2
    wid = lax.axis_index("s") * nc + lax.axis_index("c")

    iot = lax.iota(jnp.int32, 16)
    idx_q2rep = iot & 7
    idx_qp = [8 + 2 * u + (iot >> 3) for u in range(4)]
    idx_p = [iot * 0 + i for i in range(8)]

    def fetch(j, pbuf, psem):
        g = j * _NW + wid
        pltpu.make_async_copy(
            par_hbm.at[pl.ds(g * _CD, _CD)], pbuf, psem).start()

    def do_chunk(j, pbuf, psem, obuf, sem):
        g = j * _NW + wid
        pltpu.make_async_copy(
            par_hbm.at[pl.ds(0, _CD)], pbuf, psem).wait()

        @plsc.parallel_loop(0, _CD, 1, unroll=8)
        def _depo(d):
            va = pbuf[d, pl.ds(0, 16)]
            vb = pbuf[d, pl.ds(16, 16)]
            q2rep = _vgather(vb, idx_q2rep)
            qp = [_vgather(va, idx_qp[u]) * q2rep for u in range(4)]
            for i in range(8):
                p = _vgather(va, idx_p[i])
                for u in range(4):
                    obuf[d, pl.ds(i * 64 + u * 16, 16)] = p * qp[u]
        pltpu.make_async_copy(
            obuf, out_hbm.at[pl.ds(g * _CD, _CD)], sem).start()

    fetch(0, pbuf0, psem0)
    fetch(1, pbuf1, psem1)

    def chunk_pair(p, carry):
        @pl.when(p >= 1)
        def _():
            pltpu.make_async_copy(
                obuf0, out_hbm.at[pl.ds(0, _CD)], sem0).wait()
            pltpu.make_async_copy(
                obuf1, out_hbm.at[pl.ds(0, _CD)], sem1).wait()
        do_chunk(p * 2, pbuf0, psem0, obuf0, sem0)

        @pl.when(p * 2 + 2 <= _JFULL - 1)
        def _():
            fetch(p * 2 + 2, pbuf0, psem0)

        @pl.when((p * 2 + 2 == _JFULL) & (wid < _EXTRA))
        def _():
            fetch(_JFULL, pbuf0, psem0)

        do_chunk(p * 2 + 1, pbuf1, psem1, obuf1, sem1)

        @pl.when(p * 2 + 3 <= _JFULL - 1)
        def _():
            fetch(p * 2 + 3, pbuf1, psem1)
        return carry

    lax.fori_loop(0, _JFULL // 2, chunk_pair, 0, unroll=1)

    @pl.when(wid < _EXTRA)
    def _():
        pltpu.make_async_copy(
            obuf0, out_hbm.at[pl.ds(0, _CD)], sem0).wait()
        do_chunk(_JFULL, pbuf0, psem0, obuf0, sem0)

    pltpu.make_async_copy(
        obuf0, out_hbm.at[pl.ds(0, _CD)], sem0).wait()
    pltpu.make_async_copy(
        obuf1, out_hbm.at[pl.ds(0, _CD)], sem1).wait()


def _sc_raster(params_flat, n):
    mesh = plsc.VectorSubcoreMesh(core_axis_name="c", subcore_axis_name="s")
    kern = functools.partial(
        pl.kernel,
        out_type=jax.ShapeDtypeStruct((n, 512), jnp.float32),
        mesh=mesh,
        scratch_types=[
            pltpu.VMEM((_CD, 32), jnp.float32),
            pltpu.VMEM((_CD, 32), jnp.float32),
            pltpu.VMEM((_CD, 512), jnp.float32),
            pltpu.VMEM((_CD, 512), jnp.float32),
            pltpu.SemaphoreType.DMA,
            pltpu.SemaphoreType.DMA,
            pltpu.SemaphoreType.DMA,
            pltpu.SemaphoreType.DMA,
        ],
    )(_sc_raster_body)
    return kern(params_flat)


def kernel(sigma, time, charge, tail, grid_spacing, velocity):
    n = sigma.shape[0]
    grid = n // _CH
    c_t = jnp.stack([tail[:, 1], tail[:, 0], time]).reshape(3, grid, _CH)
    c_t = c_t.transpose(1, 0, 2)
    s_t = sigma.T.reshape(3, grid, _CH).transpose(1, 0, 2)
    ch_t = charge.reshape(grid, 1, _CH)
    params, offsets_t = _tc_prep(c_t, s_t, ch_t, grid_spacing, n, grid)
    rasters = _sc_raster(params, n)
    offsets = offsets_t.transpose(1, 0, 2).reshape(3, n).T
    return rasters.reshape(n, _PATCH, _PATCH, _PATCH), offsets
